# baseline (speedup 1.0000x reference)
import functools

import jax
import jax.numpy as jnp
from jax import lax
from jax.experimental import pallas as pl
from jax.experimental.pallas import tpu as pltpu
from jax.experimental.pallas import tpu_sc as plsc

_ALPHA = 3.0
_K = 20


def _gather_rows_sc(table, idx):
    info = plsc.get_sparse_core_info()
    nc, ns = info.num_cores, info.num_subcores
    nw = nc * ns
    b, d = idx.shape[0], table.shape[1]
    b_per_w = b // nw
    mesh = plsc.VectorSubcoreMesh(core_axis_name="c", subcore_axis_name="s")

    @functools.partial(
        pl.kernel,
        mesh=mesh,
        compiler_params=pltpu.CompilerParams(use_tc_tiling_on_sc=False),
        out_type=jax.ShapeDtypeStruct((b, d), jnp.float32),
        scratch_types=[
            pltpu.VMEM((b_per_w,), jnp.int32),
            pltpu.VMEM((b_per_w, d), jnp.float32),
            pltpu.SemaphoreType.DMA,
        ],
    )
    def gather_kernel(table_hbm, idx_hbm, out_hbm, idx_v, rows_v, sem):
        wid = lax.axis_index("s") * nc + lax.axis_index("c")
        base = wid * b_per_w
        pltpu.sync_copy(idx_hbm.at[pl.ds(base, b_per_w)], idx_v)
        pltpu.async_copy(table_hbm.at[idx_v], rows_v, sem).wait()
        pltpu.sync_copy(rows_v, out_hbm.at[pl.ds(base, b_per_w)])

    return gather_kernel(table, idx)


def _h_body(x_ref, w1_ref, b1_ref, he_ref, w2_ref, b2_ref, h_ref):
    dn = (((1,), (1,)), ((), ()))
    z1 = lax.dot_general(x_ref[...], w1_ref[...], dn,
                         preferred_element_type=jnp.float32)
    nv1 = jnp.tanh(_ALPHA * (z1 + b1_ref[...]))
    z2 = lax.dot_general(he_ref[...], w2_ref[...], dn,
                         preferred_element_type=jnp.float32)
    nv2 = jnp.tanh(_ALPHA * (z2 + b2_ref[...]))
    h0 = lax.dot_general(nv1, nv2, dn, preferred_element_type=jnp.float32)
    h_ref[...] = jnp.maximum(jnp.tanh(_ALPHA * h0), 0.0)


def _adj_topk_body(hb_ref, hall_ref, out_ref, work_ref):
    blk, b = out_ref.shape
    adj = lax.dot_general(hb_ref[...], hall_ref[...], (((1,), (1,)), ((), ())),
                          preferred_element_type=jnp.float32)
    keys = lax.bitcast_convert_type(adj, jnp.int32)
    work_ref[...] = keys
    colid = lax.broadcasted_iota(jnp.int32, (blk, b), 1)

    hi0 = jnp.max(keys, axis=1, keepdims=True)
    lo0 = jnp.zeros((blk, 1), jnp.int32)

    def bisect_val(_, lh):
        lo, hi = lh
        mid = lo + ((hi - lo + 1) >> 1)
        cnt = jnp.sum((work_ref[...] >= mid).astype(jnp.int32),
                      axis=1, keepdims=True)
        ok = cnt >= _K
        return jnp.where(ok, mid, lo), jnp.where(ok, hi, mid - 1)

    t, _ = lax.fori_loop(0, 31, bisect_val, (lo0, hi0))

    need = _K - jnp.sum((work_ref[...] > t).astype(jnp.int32),
                        axis=1, keepdims=True)

    def bisect_col(_, lh):
        lo, hi = lh
        mid = (lo + hi) >> 1
        w = work_ref[...]
        g = jnp.sum(((w == t) & (colid <= mid)).astype(jnp.int32),
                    axis=1, keepdims=True)
        ok = g >= need
        return jnp.where(ok, lo, mid + 1), jnp.where(ok, mid, hi)

    cstar, _ = lax.fori_loop(
        0, 12, bisect_col,
        (jnp.zeros((blk, 1), jnp.int32), jnp.full((blk, 1), b - 1, jnp.int32)))

    w = work_ref[...]
    sel = (w > t) | ((w == t) & (colid <= cstar))
    out_ref[...] = jnp.where(sel, lax.bitcast_convert_type(w, jnp.float32),
                             0.0)


def kernel(idx, embn, embhe, W1, b1, W2, b2):
    b = idx.shape[0]
    nhedges, hedim = embhe.shape
    ndim = embn.shape[1]

    nv1_raw = _gather_rows_sc(embn, idx.astype(jnp.int32))

    blk_h = 512
    H = pl.pallas_call(
        _h_body,
        grid=(b // blk_h,),
        in_specs=[
            pl.BlockSpec((blk_h, ndim), lambda i: (i, 0)),
            pl.BlockSpec((W1.shape[0], ndim), lambda i: (0, 0)),
            pl.BlockSpec((1, W1.shape[0]), lambda i: (0, 0)),
            pl.BlockSpec((nhedges, hedim), lambda i: (0, 0)),
            pl.BlockSpec((W2.shape[0], hedim), lambda i: (0, 0)),
            pl.BlockSpec((1, W2.shape[0]), lambda i: (0, 0)),
        ],
        out_specs=pl.BlockSpec((blk_h, nhedges), lambda i: (i, 0)),
        out_shape=jax.ShapeDtypeStruct((b, nhedges), jnp.float32),
        compiler_params=pltpu.CompilerParams(
            dimension_semantics=("parallel",)),
    )(nv1_raw, W1, b1.reshape(1, -1), embhe, W2, b2.reshape(1, -1))

    blk_a = 256
    adj = pl.pallas_call(
        _adj_topk_body,
        grid=(b // blk_a,),
        in_specs=[
            pl.BlockSpec((blk_a, nhedges), lambda i: (i, 0)),
            pl.BlockSpec((b, nhedges), lambda i: (0, 0)),
        ],
        out_specs=pl.BlockSpec((blk_a, b), lambda i: (i, 0)),
        out_shape=jax.ShapeDtypeStruct((b, b), jnp.float32),
        scratch_shapes=[pltpu.VMEM((blk_a, b), jnp.int32)],
        compiler_params=pltpu.CompilerParams(
            dimension_semantics=("parallel",)),
    )(H, H)

    return adj


# speedup vs baseline: 1.0379x; 1.0379x over previous
import functools

import jax
import jax.numpy as jnp
from jax import lax
from jax.experimental import pallas as pl
from jax.experimental.pallas import tpu as pltpu
from jax.experimental.pallas import tpu_sc as plsc

_ALPHA = 3.0
_K = 20


def _gather_rows_sc(table, idx):
    info = plsc.get_sparse_core_info()
    nc, ns = info.num_cores, info.num_subcores
    nw = nc * ns
    b, d = idx.shape[0], table.shape[1]
    b_per_w = b // nw
    mesh = plsc.VectorSubcoreMesh(core_axis_name="c", subcore_axis_name="s")

    @functools.partial(
        pl.kernel,
        mesh=mesh,
        compiler_params=pltpu.CompilerParams(use_tc_tiling_on_sc=False),
        out_type=jax.ShapeDtypeStruct((b, d), jnp.float32),
        scratch_types=[
            pltpu.VMEM((b_per_w,), jnp.int32),
            pltpu.VMEM((b_per_w, d), jnp.float32),
            pltpu.SemaphoreType.DMA,
        ],
    )
    def gather_kernel(table_hbm, idx_hbm, out_hbm, idx_v, rows_v, sem):
        wid = lax.axis_index("s") * nc + lax.axis_index("c")
        base = wid * b_per_w
        pltpu.sync_copy(idx_hbm.at[pl.ds(base, b_per_w)], idx_v)
        pltpu.async_copy(table_hbm.at[idx_v], rows_v, sem).wait()
        pltpu.sync_copy(rows_v, out_hbm.at[pl.ds(base, b_per_w)])

    return gather_kernel(table, idx)


def _h_body(x_ref, w1_ref, b1_ref, he_ref, w2_ref, b2_ref, h_ref):
    dn = (((1,), (1,)), ((), ()))
    z1 = lax.dot_general(x_ref[...], w1_ref[...], dn,
                         preferred_element_type=jnp.float32)
    nv1 = jnp.tanh(_ALPHA * (z1 + b1_ref[...]))
    z2 = lax.dot_general(he_ref[...], w2_ref[...], dn,
                         preferred_element_type=jnp.float32)
    nv2 = jnp.tanh(_ALPHA * (z2 + b2_ref[...]))
    h0 = lax.dot_general(nv1, nv2, dn, preferred_element_type=jnp.float32)
    h_ref[...] = jnp.maximum(jnp.tanh(_ALPHA * h0), 0.0)


def _thresh_body(hall_ref, hb_ref, t_ref, c_ref, work_ref):
    b, blk = work_ref.shape
    adj_t = lax.dot_general(hall_ref[...], hb_ref[...],
                            (((1,), (1,)), ((), ())),
                            preferred_element_type=jnp.float32)
    keys = lax.bitcast_convert_type(adj_t, jnp.int32)
    work_ref[...] = keys
    colid = lax.broadcasted_iota(jnp.int32, (b, blk), 0)

    hi0 = jnp.max(keys, axis=0, keepdims=True)
    lo0 = jnp.zeros((1, blk), jnp.int32)

    def bisect_val(_, lh):
        lo, hi = lh
        mid = lo + ((hi - lo + 1) >> 1)
        cnt = jnp.sum((work_ref[...] >= mid).astype(jnp.int32),
                      axis=0, keepdims=True)
        ok = cnt >= _K
        return jnp.where(ok, mid, lo), jnp.where(ok, hi, mid - 1)

    t, _ = lax.fori_loop(0, 31, bisect_val, (lo0, hi0))

    need = _K - jnp.sum((work_ref[...] > t).astype(jnp.int32),
                        axis=0, keepdims=True)

    def bisect_col(_, lh):
        lo, hi = lh
        mid = (lo + hi) >> 1
        w = work_ref[...]
        g = jnp.sum(((w == t) & (colid <= mid)).astype(jnp.int32),
                    axis=0, keepdims=True)
        ok = g >= need
        return jnp.where(ok, lo, mid + 1), jnp.where(ok, mid, hi)

    cstar, _ = lax.fori_loop(
        0, 12, bisect_col,
        (jnp.zeros((1, blk), jnp.int32), jnp.full((1, blk), b - 1, jnp.int32)))

    t_ref[...] = t
    c_ref[...] = cstar


def _mask_body(hb_ref, hall_ref, t_ref, c_ref, out_ref):
    blk, b = out_ref.shape
    adj = lax.dot_general(hb_ref[...], hall_ref[...], (((1,), (1,)), ((), ())),
                          preferred_element_type=jnp.float32)
    keys = lax.bitcast_convert_type(adj, jnp.int32)
    colid = lax.broadcasted_iota(jnp.int32, (blk, b), 1)
    t = t_ref[...]
    cstar = c_ref[...]
    sel = (keys > t) | ((keys == t) & (colid <= cstar))
    out_ref[...] = jnp.where(sel, adj, 0.0)


def kernel(idx, embn, embhe, W1, b1, W2, b2):
    b = idx.shape[0]
    nhedges, hedim = embhe.shape
    ndim = embn.shape[1]

    nv1_raw = _gather_rows_sc(embn, idx.astype(jnp.int32))

    blk_h = 512
    H = pl.pallas_call(
        _h_body,
        grid=(b // blk_h,),
        in_specs=[
            pl.BlockSpec((blk_h, ndim), lambda i: (i, 0)),
            pl.BlockSpec((W1.shape[0], ndim), lambda i: (0, 0)),
            pl.BlockSpec((1, W1.shape[0]), lambda i: (0, 0)),
            pl.BlockSpec((nhedges, hedim), lambda i: (0, 0)),
            pl.BlockSpec((W2.shape[0], hedim), lambda i: (0, 0)),
            pl.BlockSpec((1, W2.shape[0]), lambda i: (0, 0)),
        ],
        out_specs=pl.BlockSpec((blk_h, nhedges), lambda i: (i, 0)),
        out_shape=jax.ShapeDtypeStruct((b, nhedges), jnp.float32),
        compiler_params=pltpu.CompilerParams(
            dimension_semantics=("parallel",)),
    )(nv1_raw, W1, b1.reshape(1, -1), embhe, W2, b2.reshape(1, -1))

    blk_a = 256
    t_row, c_row = pl.pallas_call(
        _thresh_body,
        grid=(b // blk_a,),
        in_specs=[
            pl.BlockSpec((b, nhedges), lambda i: (0, 0)),
            pl.BlockSpec((blk_a, nhedges), lambda i: (i, 0)),
        ],
        out_specs=[
            pl.BlockSpec((1, blk_a), lambda i: (0, i)),
            pl.BlockSpec((1, blk_a), lambda i: (0, i)),
        ],
        out_shape=[
            jax.ShapeDtypeStruct((1, b), jnp.int32),
            jax.ShapeDtypeStruct((1, b), jnp.int32),
        ],
        scratch_shapes=[pltpu.VMEM((b, blk_a), jnp.int32)],
        compiler_params=pltpu.CompilerParams(
            dimension_semantics=("parallel",)),
    )(H, H)

    t_col = t_row.reshape(b, 1)
    c_col = c_row.reshape(b, 1)

    adj = pl.pallas_call(
        _mask_body,
        grid=(b // blk_a,),
        in_specs=[
            pl.BlockSpec((blk_a, nhedges), lambda i: (i, 0)),
            pl.BlockSpec((b, nhedges), lambda i: (0, 0)),
            pl.BlockSpec((blk_a, 1), lambda i: (i, 0)),
            pl.BlockSpec((blk_a, 1), lambda i: (i, 0)),
        ],
        out_specs=pl.BlockSpec((blk_a, b), lambda i: (i, 0)),
        out_shape=jax.ShapeDtypeStruct((b, b), jnp.float32),
        compiler_params=pltpu.CompilerParams(
            dimension_semantics=("parallel",)),
    )(H, H, t_col, c_col)

    return adj


# speedup vs baseline: 1.5720x; 1.5146x over previous
import functools

import jax
import jax.numpy as jnp
from jax import lax
from jax.experimental import pallas as pl
from jax.experimental.pallas import tpu as pltpu
from jax.experimental.pallas import tpu_sc as plsc

_ALPHA = 3.0
_K = 20


def _gather_rows_sc(table, idx):
    info = plsc.get_sparse_core_info()
    nc, ns = info.num_cores, info.num_subcores
    nw = nc * ns
    b, d = idx.shape[0], table.shape[1]
    b_per_w = b // nw
    mesh = plsc.VectorSubcoreMesh(core_axis_name="c", subcore_axis_name="s")

    @functools.partial(
        pl.kernel,
        mesh=mesh,
        compiler_params=pltpu.CompilerParams(use_tc_tiling_on_sc=False),
        out_type=jax.ShapeDtypeStruct((b, d), jnp.float32),
        scratch_types=[
            pltpu.VMEM((b_per_w,), jnp.int32),
            pltpu.VMEM((b_per_w, d), jnp.float32),
            pltpu.SemaphoreType.DMA,
        ],
    )
    def gather_kernel(table_hbm, idx_hbm, out_hbm, idx_v, rows_v, sem):
        wid = lax.axis_index("s") * nc + lax.axis_index("c")
        base = wid * b_per_w
        pltpu.sync_copy(idx_hbm.at[pl.ds(base, b_per_w)], idx_v)
        pltpu.async_copy(table_hbm.at[idx_v], rows_v, sem).wait()
        pltpu.sync_copy(rows_v, out_hbm.at[pl.ds(base, b_per_w)])

    return gather_kernel(table, idx)


def _h_body(x_ref, w1_ref, b1_ref, he_ref, w2_ref, b2_ref, h_ref):
    dn = (((1,), (1,)), ((), ()))
    z1 = lax.dot_general(x_ref[...], w1_ref[...], dn,
                         preferred_element_type=jnp.float32)
    nv1 = jnp.tanh(_ALPHA * (z1 + b1_ref[...]))
    z2 = lax.dot_general(he_ref[...], w2_ref[...], dn,
                         preferred_element_type=jnp.float32)
    nv2 = jnp.tanh(_ALPHA * (z2 + b2_ref[...]))
    h0 = lax.dot_general(nv1, nv2, dn, preferred_element_type=jnp.float32)
    h_ref[...] = jnp.maximum(jnp.tanh(_ALPHA * h0), 0.0)


def _thresh_body(hall_ref, hb_ref, t_ref, c_ref, a16_ref, w16_ref):
    b, blk = a16_ref.shape
    adj_t = lax.dot_general(hall_ref[...], hb_ref[...],
                            (((1,), (1,)), ((), ())),
                            preferred_element_type=jnp.float32)
    keys = lax.bitcast_convert_type(adj_t, jnp.int32)
    a16 = ((keys >> 15) - 32768).astype(jnp.int16)
    a16_ref[...] = a16
    w16_ref[...] = (keys & 0x7FFF).astype(jnp.int16)
    def count16(ind16):
        x = ind16
        n = x.shape[0]
        while n > 16:
            n //= 2
            x = x[:n] + x[n:]
        return jnp.sum(x.astype(jnp.int32), axis=0, keepdims=True)

    def count_ge(ref, pivot_row):
        p16 = pivot_row.astype(jnp.int16)
        return count16((ref[...] >= p16).astype(jnp.int16))

    hi0 = (jnp.max(keys, axis=0, keepdims=True) >> 15) - 32768
    lo0 = jnp.full((1, blk), -32768, jnp.int32)

    def bisect_a(_, lh):
        lo, hi = lh
        mid = lo + ((hi - lo + 1) >> 1)
        ok = count_ge(a16_ref, mid) >= _K
        return jnp.where(ok, mid, lo), jnp.where(ok, hi, mid - 1)

    p_top, _ = lax.fori_loop(0, 16, bisect_a, (lo0, hi0))

    p16 = p_top.astype(jnp.int16)
    cnt_gt_band = count16((a16_ref[...] > p16).astype(jnp.int16))
    kp = _K - cnt_gt_band

    w16_ref[...] = jnp.where(a16_ref[...] == p16, w16_ref[...],
                             jnp.int16(-1))

    def bisect_b(_, lh):
        lo, hi = lh
        mid = lo + ((hi - lo + 1) >> 1)
        ok = count_ge(w16_ref, mid) >= kp
        return jnp.where(ok, mid, lo), jnp.where(ok, hi, mid - 1)

    low15, _ = lax.fori_loop(
        0, 15, bisect_b,
        (jnp.zeros((1, blk), jnp.int32), jnp.full((1, blk), 32767, jnp.int32)))

    t32 = ((p_top + 32768) << 15) | low15
    l16 = low15.astype(jnp.int16)
    cnt_gt_ib = count16((w16_ref[...] > l16).astype(jnp.int16))
    need = kp - cnt_gt_ib

    col16 = lax.broadcasted_iota(jnp.int32, (b, blk), 0).astype(jnp.int16)
    w16_ref[...] = jnp.where(w16_ref[...] == l16, col16, jnp.int16(32767))

    def bisect_c(_, lh):
        lo, hi = lh
        mid = (lo + hi) >> 1
        g = count16((w16_ref[...] <= mid.astype(jnp.int16)).astype(jnp.int16))
        ok = g >= need
        return jnp.where(ok, lo, mid + 1), jnp.where(ok, mid, hi)

    cstar, _ = lax.fori_loop(
        0, 12, bisect_c,
        (jnp.zeros((1, blk), jnp.int32), jnp.full((1, blk), b - 1, jnp.int32)))

    t_ref[...] = t32
    c_ref[...] = cstar


def _mask_body(hb_ref, hall_ref, t_ref, c_ref, out_ref):
    blk, b = out_ref.shape
    adj = lax.dot_general(hb_ref[...], hall_ref[...], (((1,), (1,)), ((), ())),
                          preferred_element_type=jnp.float32)
    keys = lax.bitcast_convert_type(adj, jnp.int32)
    colid = lax.broadcasted_iota(jnp.int32, (blk, b), 1)
    t = t_ref[...]
    cstar = c_ref[...]
    sel = (keys > t) | ((keys == t) & (colid <= cstar))
    out_ref[...] = jnp.where(sel, adj, 0.0)


def kernel(idx, embn, embhe, W1, b1, W2, b2):
    b = idx.shape[0]
    nhedges, hedim = embhe.shape
    ndim = embn.shape[1]

    nv1_raw = _gather_rows_sc(embn, idx.astype(jnp.int32))

    blk_h = 512
    H = pl.pallas_call(
        _h_body,
        grid=(b // blk_h,),
        in_specs=[
            pl.BlockSpec((blk_h, ndim), lambda i: (i, 0)),
            pl.BlockSpec((W1.shape[0], ndim), lambda i: (0, 0)),
            pl.BlockSpec((1, W1.shape[0]), lambda i: (0, 0)),
            pl.BlockSpec((nhedges, hedim), lambda i: (0, 0)),
            pl.BlockSpec((W2.shape[0], hedim), lambda i: (0, 0)),
            pl.BlockSpec((1, W2.shape[0]), lambda i: (0, 0)),
        ],
        out_specs=pl.BlockSpec((blk_h, nhedges), lambda i: (i, 0)),
        out_shape=jax.ShapeDtypeStruct((b, nhedges), jnp.float32),
        compiler_params=pltpu.CompilerParams(
            dimension_semantics=("parallel",)),
    )(nv1_raw, W1, b1.reshape(1, -1), embhe, W2, b2.reshape(1, -1))

    blk_a = 256
    t_row, c_row = pl.pallas_call(
        _thresh_body,
        grid=(b // blk_a,),
        in_specs=[
            pl.BlockSpec((b, nhedges), lambda i: (0, 0)),
            pl.BlockSpec((blk_a, nhedges), lambda i: (i, 0)),
        ],
        out_specs=[
            pl.BlockSpec((1, blk_a), lambda i: (0, i)),
            pl.BlockSpec((1, blk_a), lambda i: (0, i)),
        ],
        out_shape=[
            jax.ShapeDtypeStruct((1, b), jnp.int32),
            jax.ShapeDtypeStruct((1, b), jnp.int32),
        ],
        scratch_shapes=[pltpu.VMEM((b, blk_a), jnp.int16),
                        pltpu.VMEM((b, blk_a), jnp.int16)],
        compiler_params=pltpu.CompilerParams(
            dimension_semantics=("parallel",)),
    )(H, H)

    t_col = t_row.reshape(b, 1)
    c_col = c_row.reshape(b, 1)

    adj = pl.pallas_call(
        _mask_body,
        grid=(b // blk_a,),
        in_specs=[
            pl.BlockSpec((blk_a, nhedges), lambda i: (i, 0)),
            pl.BlockSpec((b, nhedges), lambda i: (0, 0)),
            pl.BlockSpec((blk_a, 1), lambda i: (i, 0)),
            pl.BlockSpec((blk_a, 1), lambda i: (i, 0)),
        ],
        out_specs=pl.BlockSpec((blk_a, b), lambda i: (i, 0)),
        out_shape=jax.ShapeDtypeStruct((b, b), jnp.float32),
        compiler_params=pltpu.CompilerParams(
            dimension_semantics=("parallel",)),
    )(H, H, t_col, c_col)

    return adj
